# baseline (speedup 1.0000x reference)
import functools

import jax
import jax.numpy as jnp
from jax import lax
from jax.experimental import pallas as pl
from jax.experimental.pallas import tpu as pltpu
from jax.experimental.pallas import tpu_sc as plsc

_SC_NC = 2
_SC_NS = 16
_SC_NW = _SC_NC * _SC_NS


def _sc_route(table, dest, sort_idx, n_out, prob=None,
              combine=False, chunk=64, nbuf=2):
    V, D = table.shape
    P = sort_idx.shape[0]
    assert P % (_SC_NW * chunk) == 0, (P, chunk)
    p_per_w = P // _SC_NW
    nch = p_per_w // chunk
    dest = dest.reshape(_SC_NW, nch, chunk)
    sort_idx = sort_idx.reshape(_SC_NW, nch, chunk)
    have_prob = prob is not None
    extra_in = (prob.reshape(_SC_NW, nch, chunk),) if have_prob else ()
    out_type = [jax.ShapeDtypeStruct((n_out, D), table.dtype)]
    if have_prob:
        out_type.append(jax.ShapeDtypeStruct((n_out,), jnp.float32))
    mesh = plsc.VectorSubcoreMesh(
        core_axis_name="c", subcore_axis_name="s",
        num_cores=_SC_NC, num_subcores=_SC_NS)

    @functools.partial(
        pl.kernel, mesh=mesh,
        out_type=out_type,
        scratch_types=(
            [pltpu.VMEM((nch, chunk), jnp.int32),
             pltpu.VMEM((nch, chunk), jnp.int32)]
            + ([pltpu.VMEM((nch, chunk), jnp.float32)] if have_prob else [])
            + [pltpu.VMEM((chunk, D), table.dtype) for _ in range(nbuf)]
            + [pltpu.SemaphoreType.DMA for _ in range(nbuf)]
            + [pltpu.SemaphoreType.DMA for _ in range(nbuf)]
            + [pltpu.SemaphoreType.DMA]
        ),
    )
    def k(table_hbm, de_hbm, si_hbm, *rest):
        if have_prob:
            prob_hbm, out_hbm, pp_hbm = rest[0], rest[1], rest[2]
            rest = rest[3:]
        else:
            out_hbm = rest[0]
            rest = rest[1:]
        si_v, de_v = rest[:2]
        rest = rest[2:]
        if have_prob:
            prob_v, rest = rest[0], rest[1:]
        rows = rest[:nbuf]
        gsems = rest[nbuf:2 * nbuf]
        wsems = rest[2 * nbuf:3 * nbuf]
        psem = rest[3 * nbuf]
        wid = lax.axis_index("s") * _SC_NC + lax.axis_index("c")
        pltpu.sync_copy(si_hbm.at[wid], si_v)
        pltpu.sync_copy(de_hbm.at[wid], de_v)
        if have_prob:
            pltpu.sync_copy(prob_hbm.at[wid], prob_v)
        iidx, oidx = (de_v, si_v) if combine else (si_v, de_v)
        gd = [None] * nbuf
        wd = [None] * nbuf
        for b in range(min(nbuf, nch)):
            gd[b] = pltpu.async_copy(
                table_hbm.at[iidx.at[b]], rows[b], gsems[b])
        for c in range(nch):
            b = c % nbuf
            gd[b].wait()
            wd[b] = pltpu.async_copy(
                rows[b], out_hbm.at[oidx.at[c]], wsems[b])
            if have_prob:
                pltpu.async_copy(prob_v.at[c], pp_hbm.at[de_v.at[c]], psem)
            nxt = c + nbuf
            if nxt < nch:
                wd[b].wait()
                gd[b] = pltpu.async_copy(
                    table_hbm.at[iidx.at[nxt]], rows[b], gsems[b])
        for b in range(min(nbuf, nch)):
            if wd[b] is not None:
                wd[b].wait()
        if have_prob:
            for c in range(nch):
                pltpu.make_async_copy(
                    prob_v.at[c], pp_hbm.at[de_v.at[c]], psem).wait()

    return k(table, dest, sort_idx, *extra_in)


def _gate_body(x_ref, wg_ref, gate_ref, psel_ref, psum_ref, cnt_ref):
    x = x_ref[...]
    wg = wg_ref[...]
    logits = jax.lax.dot_general(
        x, wg, (((1,), (1,)), ((), ())),
        preferred_element_type=jnp.float32)
    m = jnp.max(logits, axis=-1, keepdims=True)
    p = jnp.exp(logits - m)
    s = jnp.sum(p, axis=-1)
    g = jnp.argmax(logits, axis=-1).astype(jnp.int32)
    prob = p / s[:, None]
    e_iota = jax.lax.broadcasted_iota(jnp.int32, prob.shape, 1)
    onehot = (g[:, None] == e_iota)
    gate_ref[0, 0, :] = g
    psel_ref[0, 0, :] = 1.0 / s
    psum_ref[0, 0, :] = jnp.sum(prob, axis=0)
    cnt_ref[0, 0, :] = jnp.sum(onehot.astype(jnp.int32), axis=0)


def _expert_body(bexp_ref, nact_ref, x_ref, w_ref, b_ref, p_ref, o_ref):
    i = pl.program_id(0)

    @pl.when(i < nact_ref[0])
    def _():
        y = jax.lax.dot_general(
            x_ref[...], w_ref[0], (((1,), (1,)), ((), ())),
            preferred_element_type=jnp.float32)
        y = y + b_ref[0]
        o_ref[...] = y * p_ref[0, 0, :][:, None]


@functools.partial(jax.jit, static_argnames=())
def kernel(x, Wg, We, be):
    bsz, seq_len, D = x.shape
    T = bsz * seq_len
    E = Wg.shape[0]
    xf = x.reshape(T, D)

    TB = 1024
    GB = T // TB
    gate_b, psel_b, psum_b, cnt_b = pl.pallas_call(
        _gate_body,
        grid=(GB,),
        in_specs=[
            pl.BlockSpec((TB, D), lambda i: (i, 0)),
            pl.BlockSpec((E, D), lambda i: (0, 0)),
        ],
        out_specs=[
            pl.BlockSpec((1, 1, TB), lambda i: (i, 0, 0)),
            pl.BlockSpec((1, 1, TB), lambda i: (i, 0, 0)),
            pl.BlockSpec((1, 1, E), lambda i: (i, 0, 0)),
            pl.BlockSpec((1, 1, E), lambda i: (i, 0, 0)),
        ],
        out_shape=[
            jax.ShapeDtypeStruct((GB, 1, TB), jnp.int32),
            jax.ShapeDtypeStruct((GB, 1, TB), jnp.float32),
            jax.ShapeDtypeStruct((GB, 1, E), jnp.float32),
            jax.ShapeDtypeStruct((GB, 1, E), jnp.int32),
        ],
    )(xf, Wg)
    gate = gate_b.reshape(T)
    prob_sel = psel_b.reshape(T)
    counts = jnp.sum(cnt_b, axis=(0, 1))
    P = jnp.sum(psum_b, axis=(0, 1)) / T
    f = counts.astype(jnp.float32) / T
    balance_loss = E * jnp.sum(P * f)

    B = 512
    NB = T // B + E
    prank = jnp.arange(T, dtype=jnp.int32)
    gate_sorted, sort_idx, prob_sorted = jax.lax.sort(
        (gate, prank, prob_sel), num_keys=1)
    bpe = (counts + B - 1) // B
    bpe_cum = jnp.cumsum(bpe)
    block_start = bpe_cum - bpe
    nb_active = bpe_cum[-1].astype(jnp.int32).reshape(1)
    block_expert = jnp.minimum(
        jnp.searchsorted(bpe_cum, jnp.arange(NB, dtype=jnp.int32),
                         side="right", method="compare_all"),
        E - 1).astype(jnp.int32)
    expert_start = jnp.cumsum(counts) - counts
    pad_offset = (B * block_start - expert_start).astype(jnp.int32)
    dest = prank + pad_offset[gate_sorted]

    xs, prob_pad = _sc_route(xf, dest, sort_idx, NB * B, prob=prob_sorted)
    prob_pad = prob_pad.reshape(NB, 1, B)

    grid_spec = pltpu.PrefetchScalarGridSpec(
        num_scalar_prefetch=2,
        grid=(NB,),
        in_specs=[
            pl.BlockSpec((B, D), lambda i, bexp, nact: (i, 0)),
            pl.BlockSpec((1, D, D), lambda i, bexp, nact: (bexp[i], 0, 0)),
            pl.BlockSpec((1, 1, D), lambda i, bexp, nact: (bexp[i], 0, 0)),
            pl.BlockSpec((1, 1, B), lambda i, bexp, nact: (i, 0, 0)),
        ],
        out_specs=pl.BlockSpec((B, D), lambda i, bexp, nact: (i, 0)),
    )
    ys = pl.pallas_call(
        _expert_body,
        grid_spec=grid_spec,
        out_shape=jax.ShapeDtypeStruct((NB * B, D), jnp.float32),
    )(block_expert, nb_active, xs, We, be.reshape(E, 1, D), prob_pad)

    out = _sc_route(ys, dest, sort_idx, T, combine=True)
    out = out[0].reshape(bsz, seq_len, D)
    return out, balance_loss, counts


# speedup vs baseline: 1.1734x; 1.1734x over previous
import functools

import jax
import jax.numpy as jnp
from jax import lax
from jax.experimental import pallas as pl
from jax.experimental.pallas import tpu as pltpu
from jax.experimental.pallas import tpu_sc as plsc

_SC_NC = 2
_SC_NS = 16
_SC_NW = _SC_NC * _SC_NS


def _sc_route(table, dest, sort_idx, n_out, prob=None,
              combine=False, chunk=64, nbuf=2):
    V, D = table.shape
    P = sort_idx.shape[0]
    assert P % (_SC_NW * chunk) == 0, (P, chunk)
    p_per_w = P // _SC_NW
    nch = p_per_w // chunk
    dest = dest.reshape(_SC_NW, nch, chunk)
    sort_idx = sort_idx.reshape(_SC_NW, nch, chunk)
    have_prob = prob is not None
    extra_in = (prob.reshape(_SC_NW, nch, chunk),) if have_prob else ()
    out_type = [jax.ShapeDtypeStruct((n_out, D), table.dtype)]
    if have_prob:
        out_type.append(jax.ShapeDtypeStruct((n_out,), jnp.float32))
    mesh = plsc.VectorSubcoreMesh(
        core_axis_name="c", subcore_axis_name="s",
        num_cores=_SC_NC, num_subcores=_SC_NS)

    @functools.partial(
        pl.kernel, mesh=mesh,
        out_type=out_type,
        scratch_types=(
            [pltpu.VMEM((nch, chunk), jnp.int32),
             pltpu.VMEM((nch, chunk), jnp.int32)]
            + ([pltpu.VMEM((nch, chunk), jnp.float32)] if have_prob else [])
            + [pltpu.VMEM((chunk, D), table.dtype) for _ in range(nbuf)]
            + [pltpu.SemaphoreType.DMA for _ in range(nbuf)]
            + [pltpu.SemaphoreType.DMA for _ in range(nbuf)]
            + [pltpu.SemaphoreType.DMA]
        ),
    )
    def k(table_hbm, de_hbm, si_hbm, *rest):
        if have_prob:
            prob_hbm, out_hbm, pp_hbm = rest[0], rest[1], rest[2]
            rest = rest[3:]
        else:
            out_hbm = rest[0]
            rest = rest[1:]
        si_v, de_v = rest[:2]
        rest = rest[2:]
        if have_prob:
            prob_v, rest = rest[0], rest[1:]
        rows = rest[:nbuf]
        gsems = rest[nbuf:2 * nbuf]
        wsems = rest[2 * nbuf:3 * nbuf]
        psem = rest[3 * nbuf]
        wid = lax.axis_index("s") * _SC_NC + lax.axis_index("c")
        pltpu.sync_copy(si_hbm.at[wid], si_v)
        pltpu.sync_copy(de_hbm.at[wid], de_v)
        if have_prob:
            pltpu.sync_copy(prob_hbm.at[wid], prob_v)
        iidx, oidx = (de_v, si_v) if combine else (si_v, de_v)
        gd = [None] * nbuf
        wd = [None] * nbuf
        for b in range(min(nbuf, nch)):
            gd[b] = pltpu.async_copy(
                table_hbm.at[iidx.at[b]], rows[b], gsems[b])
        for c in range(nch):
            b = c % nbuf
            gd[b].wait()
            wd[b] = pltpu.async_copy(
                rows[b], out_hbm.at[oidx.at[c]], wsems[b])
            if have_prob:
                pltpu.async_copy(prob_v.at[c], pp_hbm.at[de_v.at[c]], psem)
            nxt = c + nbuf
            if nxt < nch:
                wd[b].wait()
                gd[b] = pltpu.async_copy(
                    table_hbm.at[iidx.at[nxt]], rows[b], gsems[b])
        for b in range(min(nbuf, nch)):
            if wd[b] is not None:
                wd[b].wait()
        if have_prob:
            for c in range(nch):
                pltpu.make_async_copy(
                    prob_v.at[c], pp_hbm.at[de_v.at[c]], psem).wait()

    return k(table, dest, sort_idx, *extra_in)


def _gate_body(x_ref, wg_ref, gate_ref, psel_ref, psum_ref, cnt_ref):
    x = x_ref[...]
    wg = wg_ref[...]
    logits = jax.lax.dot_general(
        x, wg, (((1,), (1,)), ((), ())),
        preferred_element_type=jnp.float32)
    m = jnp.max(logits, axis=-1, keepdims=True)
    p = jnp.exp(logits - m)
    s = jnp.sum(p, axis=-1)
    g = jnp.argmax(logits, axis=-1).astype(jnp.int32)
    prob = p / s[:, None]
    e_iota = jax.lax.broadcasted_iota(jnp.int32, prob.shape, 1)
    onehot = (g[:, None] == e_iota)
    gate_ref[0, 0, :] = g
    psel_ref[0, 0, :] = 1.0 / s
    psum_ref[0, 0, :] = jnp.sum(prob, axis=0)
    cnt_ref[0, 0, :] = jnp.sum(onehot.astype(jnp.int32), axis=0)


def _expert_body(bexp_ref, nact_ref, x_ref, w_ref, b_ref, o_ref):
    i = pl.program_id(0)

    @pl.when(i < nact_ref[0])
    def _():
        y = jax.lax.dot_general(
            x_ref[...], w_ref[0], (((1,), (1,)), ((), ())),
            preferred_element_type=jnp.float32)
        o_ref[...] = y + b_ref[0]


@functools.partial(jax.jit, static_argnames=())
def kernel(x, Wg, We, be):
    bsz, seq_len, D = x.shape
    T = bsz * seq_len
    E = Wg.shape[0]
    xf = x.reshape(T, D)

    TB = 1024
    GB = T // TB
    gate_b, psel_b, psum_b, cnt_b = pl.pallas_call(
        _gate_body,
        grid=(GB,),
        in_specs=[
            pl.BlockSpec((TB, D), lambda i: (i, 0)),
            pl.BlockSpec((E, D), lambda i: (0, 0)),
        ],
        out_specs=[
            pl.BlockSpec((1, 1, TB), lambda i: (i, 0, 0)),
            pl.BlockSpec((1, 1, TB), lambda i: (i, 0, 0)),
            pl.BlockSpec((1, 1, E), lambda i: (i, 0, 0)),
            pl.BlockSpec((1, 1, E), lambda i: (i, 0, 0)),
        ],
        out_shape=[
            jax.ShapeDtypeStruct((GB, 1, TB), jnp.int32),
            jax.ShapeDtypeStruct((GB, 1, TB), jnp.float32),
            jax.ShapeDtypeStruct((GB, 1, E), jnp.float32),
            jax.ShapeDtypeStruct((GB, 1, E), jnp.int32),
        ],
    )(xf, Wg)
    gate = gate_b.reshape(T)
    prob_sel = psel_b.reshape(T)
    counts = jnp.sum(cnt_b, axis=(0, 1))
    P = jnp.sum(psum_b, axis=(0, 1)) / T
    f = counts.astype(jnp.float32) / T
    balance_loss = E * jnp.sum(P * f)

    B = 512
    NB = T // B + E
    prank = jnp.arange(T, dtype=jnp.int32)
    gate_sorted, sort_idx = jax.lax.sort((gate, prank), num_keys=1)
    bpe = (counts + B - 1) // B
    bpe_cum = jnp.cumsum(bpe)
    block_start = bpe_cum - bpe
    nb_active = bpe_cum[-1].astype(jnp.int32).reshape(1)
    block_expert = jnp.minimum(
        jnp.searchsorted(bpe_cum, jnp.arange(NB, dtype=jnp.int32),
                         side="right", method="compare_all"),
        E - 1).astype(jnp.int32)
    expert_start = jnp.cumsum(counts) - counts
    pad_offset = (B * block_start - expert_start).astype(jnp.int32)
    dest = prank + pad_offset[gate_sorted]

    xs, = _sc_route(xf, dest, sort_idx, NB * B)

    grid_spec = pltpu.PrefetchScalarGridSpec(
        num_scalar_prefetch=2,
        grid=(NB,),
        in_specs=[
            pl.BlockSpec((B, D), lambda i, bexp, nact: (i, 0)),
            pl.BlockSpec((1, D, D), lambda i, bexp, nact: (bexp[i], 0, 0)),
            pl.BlockSpec((1, 1, D), lambda i, bexp, nact: (bexp[i], 0, 0)),
        ],
        out_specs=pl.BlockSpec((B, D), lambda i, bexp, nact: (i, 0)),
    )
    ys = pl.pallas_call(
        _expert_body,
        grid_spec=grid_spec,
        out_shape=jax.ShapeDtypeStruct((NB * B, D), jnp.float32),
    )(block_expert, nb_active, xs, We, be.reshape(E, 1, D))

    out, = _sc_route(ys, dest, sort_idx, T, combine=True)
    out = (out * prob_sel[:, None]).reshape(bsz, seq_len, D)
    return out, balance_loss, counts
